# baseline (speedup 1.0000x reference)
import functools

import jax
import jax.numpy as jnp
from jax import lax
from jax.experimental import pallas as pl
from jax.experimental.pallas import tpu as pltpu
from jax.experimental.pallas import tpu_sc as plsc

_S = 3.0
_M = 0.2
_NEG = -3.0e38

_B, _C = 1024, 100000
_CAL = (_C // 128) * 128
_CH = 1408
_NCHUNK = _CAL // _CH
_RPW = 8


def _tc_kernel(x_ref, m_ref, s_ref):
    x = x_ref[...]
    m = jnp.max(x, axis=1, keepdims=True)
    m_ref[...] = m
    s_ref[...] = jnp.sum(jnp.exp(_S * x - _S * m), axis=1, keepdims=True)


def _tc_main(cos_theta, b_tc):
    return pl.pallas_call(
        _tc_kernel,
        grid=(b_tc // 32,),
        in_specs=[pl.BlockSpec((32, _C), lambda i: (i, 0))],
        out_specs=[
            pl.BlockSpec((32, 1), lambda i: (i, 0)),
            pl.BlockSpec((32, 1), lambda i: (i, 0)),
        ],
        out_shape=[
            jax.ShapeDtypeStruct((b_tc, 1), jnp.float32),
            jax.ShapeDtypeStruct((b_tc, 1), jnp.float32),
        ],
    )(cos_theta)


def _make_sc(b_tc):
    sc_rows = _B - b_tc
    info = plsc.get_sparse_core_info()
    nc, ns = info.num_cores, info.num_subcores
    nw = nc * ns
    g_per_w = _B // nw
    mesh = plsc.VectorSubcoreMesh(core_axis_name="c", subcore_axis_name="s")

    @functools.partial(
        pl.kernel,
        mesh=mesh,
        out_type=[
            jax.ShapeDtypeStruct((sc_rows, 16), jnp.float32),
            jax.ShapeDtypeStruct((sc_rows, 16), jnp.float32),
            jax.ShapeDtypeStruct((_B, 16), jnp.float32),
        ],
        scratch_types=[
            pltpu.VMEM((8, _CH), jnp.float32),
            pltpu.VMEM((8, 16), jnp.float32),
            pltpu.VMEM((8, 16), jnp.float32),
            pltpu.VMEM((16,), jnp.float32),
            pltpu.VMEM((g_per_w,), jnp.int32),
            pltpu.VMEM((128,), jnp.float32),
            pltpu.VMEM((g_per_w, 16), jnp.float32),
        ],
    )
    def sc_kernel(x_hbm, t_hbm, partm_hbm, parts_hbm, tv_hbm, buf, m_ref, s_ref,
                  res_ref, tbuf, gbuf, wout):
        wid = lax.axis_index("s") * nc + lax.axis_index("c")

        gbase = pl.multiple_of(wid * g_per_w, 8)
        pltpu.sync_copy(t_hbm.at[pl.ds(gbase, g_per_w)], tbuf)

        for r in range(g_per_w):
            t = tbuf[pl.ds((r // 16) * 16, 16)][r % 16]
            c0 = pl.multiple_of((t // 128) * 128, 128)
            pltpu.sync_copy(x_hbm.at[gbase + r, pl.ds(c0, 128)], gbuf)
            j16 = (t - c0) // 16
            w = jnp.zeros((16,), jnp.float32)
            for j in range(8):
                w = jnp.where(j16 == j, gbuf[pl.ds(j * 16, 16)], w)
            wout[r, :] = w
        pltpu.sync_copy(wout, tv_hbm.at[pl.ds(gbase, g_per_w)])

        grow = pl.multiple_of(b_tc + wid * _RPW, 8)
        m_ref[...] = jnp.full((8, 16), _NEG, jnp.float32)
        s_ref[...] = jnp.zeros((8, 16), jnp.float32)
        def chunk_body(k, carry):
            co = pl.multiple_of(k * _CH, 128)
            pltpu.sync_copy(x_hbm.at[pl.ds(grow, 8), pl.ds(co, _CH)], buf)
            for r in range(8):
                def vmax_body(j, mv, r=r):
                    return jnp.maximum(mv, buf[r, pl.ds(j * 16, 16)])

                cm = lax.fori_loop(
                    0, _CH // 16, vmax_body, jnp.full((16,), _NEG, jnp.float32)
                )

                def vsum_body(j, sv, r=r, cm=cm):
                    v = buf[r, pl.ds(j * 16, 16)]
                    return sv + jnp.exp(_S * v - _S * cm)

                cs = lax.fori_loop(
                    0, _CH // 16, vsum_body, jnp.zeros((16,), jnp.float32)
                )
                m_old = m_ref[r, :]
                m_new = jnp.maximum(m_old, cm)
                s_ref[r, :] = s_ref[r, :] * jnp.exp(_S * (m_old - m_new)) + cs * jnp.exp(
                    _S * (cm - m_new)
                )
                m_ref[r, :] = m_new
            return carry

        lax.fori_loop(0, _NCHUNK, chunk_body, 0)

        for r in range(8):
            res_ref[...] = m_ref[r, :]
            pltpu.sync_copy(res_ref, partm_hbm.at[wid * _RPW + r])
            res_ref[...] = s_ref[r, :]
            pltpu.sync_copy(res_ref, parts_hbm.at[wid * _RPW + r])

    return sc_kernel


def _combine_kernel(mtc_ref, stc_ref, scpm_ref, scps_ref, tail_ref, tw_ref,
                    t_ref, out_ref, *, b_tc, sc_rows):
    tw = tw_ref[...]
    tmod = t_ref[...] % 16
    lane16 = jax.lax.broadcasted_iota(jnp.int32, (_B, 16), 1)
    tv = jnp.sum(jnp.where(lane16 == tmod, tw, 0.0), axis=1, keepdims=True)
    out_t = _S * tv - _S * _M

    m1 = mtc_ref[...]
    s1 = stc_ref[...]
    tv1 = tv[:b_tc, :]
    ot1 = out_t[:b_tc, :]
    sc1 = s1 - jnp.exp(_S * tv1 - _S * m1) + jnp.exp(ot1 - _S * m1)
    loss1 = _S * m1 + jnp.log(sc1) - ot1

    m_l = scpm_ref[...]
    s_l = scps_ref[...]
    m2 = jnp.max(m_l, axis=1, keepdims=True)
    s2 = jnp.sum(s_l * jnp.exp(_S * (m_l - m2)), axis=1, keepdims=True)
    xt = tail_ref[...]
    cols = jax.lax.broadcasted_iota(jnp.int32, (sc_rows, 128), 1)
    xt = jnp.where(cols < _C - _CAL, xt, -jnp.inf)
    tm = jnp.max(xt, axis=1, keepdims=True)
    ts = jnp.sum(jnp.exp(_S * xt - _S * tm), axis=1, keepdims=True)
    mf = jnp.maximum(m2, tm)
    sf = s2 * jnp.exp(_S * (m2 - mf)) + ts * jnp.exp(_S * (tm - mf))
    tv2 = tv[b_tc:, :]
    ot2 = out_t[b_tc:, :]
    sc2 = sf - jnp.exp(_S * tv2 - _S * mf) + jnp.exp(ot2 - _S * mf)
    loss2 = _S * mf + jnp.log(sc2) - ot2

    out_ref[...] = ((jnp.sum(loss1) + jnp.sum(loss2)) / _B).reshape(1, 1)


def kernel(cos_theta, cos_theta_aux, target):
    B, C = cos_theta.shape
    b_tc = _B - 32 * _RPW
    sc_rows = B - b_tc

    t32 = target.astype(jnp.int32)
    m_tc, s_tc = _tc_main(cos_theta, b_tc)
    scpm, scps, tw = _make_sc(b_tc)(cos_theta, t32)

    out = pl.pallas_call(
        functools.partial(_combine_kernel, b_tc=b_tc, sc_rows=sc_rows),
        grid=(1,),
        in_specs=[
            pl.BlockSpec((b_tc, 1), lambda i: (0, 0)),
            pl.BlockSpec((b_tc, 1), lambda i: (0, 0)),
            pl.BlockSpec((sc_rows, 16), lambda i: (0, 0)),
            pl.BlockSpec((sc_rows, 16), lambda i: (0, 0)),
            pl.BlockSpec((sc_rows, 128), lambda i: (b_tc // sc_rows, _CAL // 128)),
            pl.BlockSpec((B, 16), lambda i: (0, 0)),
            pl.BlockSpec((B, 1), lambda i: (0, 0)),
        ],
        out_specs=pl.BlockSpec((1, 1), lambda i: (0, 0)),
        out_shape=jax.ShapeDtypeStruct((1, 1), jnp.float32),
    )(m_tc, s_tc, scpm, scps, cos_theta, tw, t32.reshape(B, 1))
    return out[0, 0]


# speedup vs baseline: 1.6257x; 1.6257x over previous
import functools

import jax
import jax.numpy as jnp
from jax import lax
from jax.experimental import pallas as pl
from jax.experimental.pallas import tpu as pltpu
from jax.experimental.pallas import tpu_sc as plsc

_S = 3.0
_M = 0.2

_B, _C = 1024, 100000


def _tc_kernel(x_ref, m_ref, s_ref):
    x = x_ref[...]
    m = jnp.max(x, axis=1, keepdims=True)
    m_ref[...] = m
    s_ref[...] = jnp.sum(jnp.exp(_S * x - _S * m), axis=1, keepdims=True)


def _tc_main(cos_theta):
    return pl.pallas_call(
        _tc_kernel,
        grid=(_B // 32,),
        in_specs=[pl.BlockSpec((32, _C), lambda i: (i, 0))],
        out_specs=[
            pl.BlockSpec((32, 1), lambda i: (i, 0)),
            pl.BlockSpec((32, 1), lambda i: (i, 0)),
        ],
        out_shape=[
            jax.ShapeDtypeStruct((_B, 1), jnp.float32),
            jax.ShapeDtypeStruct((_B, 1), jnp.float32),
        ],
    )(cos_theta)


def _make_sc():
    info = plsc.get_sparse_core_info()
    nc, ns = info.num_cores, info.num_subcores
    nw = nc * ns
    g_per_w = _B // nw
    mesh = plsc.VectorSubcoreMesh(core_axis_name="c", subcore_axis_name="s")

    @functools.partial(
        pl.kernel,
        mesh=mesh,
        out_type=jax.ShapeDtypeStruct((_B, 16), jnp.float32),
        scratch_types=[
            pltpu.VMEM((g_per_w,), jnp.int32),
            pltpu.VMEM((128,), jnp.float32),
            pltpu.VMEM((g_per_w, 16), jnp.float32),
        ],
    )
    def sc_kernel(x_hbm, t_hbm, tv_hbm, tbuf, gbuf, wout):
        wid = lax.axis_index("s") * nc + lax.axis_index("c")
        gbase = pl.multiple_of(wid * g_per_w, 8)
        pltpu.sync_copy(t_hbm.at[pl.ds(gbase, g_per_w)], tbuf)

        for r in range(g_per_w):
            t = tbuf[pl.ds((r // 16) * 16, 16)][r % 16]
            c0 = pl.multiple_of((t // 128) * 128, 128)
            pltpu.sync_copy(x_hbm.at[gbase + r, pl.ds(c0, 128)], gbuf)
            j16 = (t - c0) // 16
            w = jnp.zeros((16,), jnp.float32)
            for j in range(8):
                w = jnp.where(j16 == j, gbuf[pl.ds(j * 16, 16)], w)
            wout[r, :] = w
        pltpu.sync_copy(wout, tv_hbm.at[pl.ds(gbase, g_per_w)])

    return sc_kernel


def _combine_kernel(m_ref, s_ref, tw_ref, t_ref, out_ref):
    tw = tw_ref[...]
    tmod = t_ref[...] % 16
    lane16 = jax.lax.broadcasted_iota(jnp.int32, (_B, 16), 1)
    tv = jnp.sum(jnp.where(lane16 == tmod, tw, 0.0), axis=1, keepdims=True)

    m = m_ref[...]
    s = s_ref[...]
    out_t = _S * tv - _S * _M
    s_c = s - jnp.exp(_S * tv - _S * m) + jnp.exp(out_t - _S * m)
    loss = _S * m + jnp.log(s_c) - out_t
    out_ref[...] = (jnp.sum(loss) / _B).reshape(1, 1)


def kernel(cos_theta, cos_theta_aux, target):
    B, C = cos_theta.shape
    t32 = target.astype(jnp.int32)
    m_tc, s_tc = _tc_main(cos_theta)
    tw = _make_sc()(cos_theta, t32)

    out = pl.pallas_call(
        _combine_kernel,
        grid=(1,),
        in_specs=[
            pl.BlockSpec((B, 1), lambda i: (0, 0)),
            pl.BlockSpec((B, 1), lambda i: (0, 0)),
            pl.BlockSpec((B, 16), lambda i: (0, 0)),
            pl.BlockSpec((B, 1), lambda i: (0, 0)),
        ],
        out_specs=pl.BlockSpec((1, 1), lambda i: (0, 0)),
        out_shape=jax.ShapeDtypeStruct((1, 1), jnp.float32),
    )(m_tc, s_tc, tw, t32.reshape(B, 1))
    return out[0, 0]


# speedup vs baseline: 1.6366x; 1.0067x over previous
import functools

import jax
import jax.numpy as jnp
from jax import lax
from jax.experimental import pallas as pl
from jax.experimental.pallas import tpu as pltpu
from jax.experimental.pallas import tpu_sc as plsc

_S = 3.0
_M = 0.2

_B, _C = 1024, 100000


def _tc_kernel(x_ref, m_ref, s_ref, *, nblk, blk):
    k = pl.program_id(0)

    @pl.when(k == 0)
    def _init():
        m_ref[...] = jnp.full((_B, 1), -jnp.inf, jnp.float32)
        s_ref[...] = jnp.zeros((_B, 1), jnp.float32)

    def update(x):
        bm = jnp.max(x, axis=1, keepdims=True)
        bs = jnp.sum(jnp.exp(_S * x - _S * bm), axis=1, keepdims=True)
        m_old = m_ref[...]
        m_new = jnp.maximum(m_old, bm)
        s_ref[...] = s_ref[...] * jnp.exp(_S * (m_old - m_new)) + bs * jnp.exp(
            _S * (bm - m_new)
        )
        m_ref[...] = m_new

    @pl.when(k < nblk - 1)
    def _full_block():
        update(x_ref[...])

    @pl.when(k == nblk - 1)
    def _tail_block():
        cols = jax.lax.broadcasted_iota(jnp.int32, (_B, blk), 1) + k * blk
        update(jnp.where(cols < _C, x_ref[...], -jnp.inf))


def _tc_main(cos_theta):
    blk = 2048
    nblk = pl.cdiv(_C, blk)
    return pl.pallas_call(
        functools.partial(_tc_kernel, nblk=nblk, blk=blk),
        grid=(nblk,),
        in_specs=[pl.BlockSpec((_B, blk), lambda k: (0, k))],
        out_specs=[
            pl.BlockSpec((_B, 1), lambda k: (0, 0)),
            pl.BlockSpec((_B, 1), lambda k: (0, 0)),
        ],
        out_shape=[
            jax.ShapeDtypeStruct((_B, 1), jnp.float32),
            jax.ShapeDtypeStruct((_B, 1), jnp.float32),
        ],
    )(cos_theta)


def _make_sc():
    info = plsc.get_sparse_core_info()
    nc, ns = info.num_cores, info.num_subcores
    nw = nc * ns
    g_per_w = _B // nw
    mesh = plsc.VectorSubcoreMesh(core_axis_name="c", subcore_axis_name="s")

    @functools.partial(
        pl.kernel,
        mesh=mesh,
        out_type=jax.ShapeDtypeStruct((_B, 16), jnp.float32),
        scratch_types=[
            pltpu.VMEM((g_per_w,), jnp.int32),
            pltpu.VMEM((128,), jnp.float32),
            pltpu.VMEM((g_per_w, 16), jnp.float32),
        ],
    )
    def sc_kernel(x_hbm, t_hbm, tv_hbm, tbuf, gbuf, wout):
        wid = lax.axis_index("s") * nc + lax.axis_index("c")
        gbase = pl.multiple_of(wid * g_per_w, 8)
        pltpu.sync_copy(t_hbm.at[pl.ds(gbase, g_per_w)], tbuf)

        for r in range(g_per_w):
            t = tbuf[pl.ds((r // 16) * 16, 16)][r % 16]
            c0 = pl.multiple_of((t // 128) * 128, 128)
            pltpu.sync_copy(x_hbm.at[gbase + r, pl.ds(c0, 128)], gbuf)
            j16 = (t - c0) // 16
            w = jnp.zeros((16,), jnp.float32)
            for j in range(8):
                w = jnp.where(j16 == j, gbuf[pl.ds(j * 16, 16)], w)
            wout[r, :] = w
        pltpu.sync_copy(wout, tv_hbm.at[pl.ds(gbase, g_per_w)])

    return sc_kernel


def _combine_kernel(m_ref, s_ref, tw_ref, t_ref, out_ref):
    tw = tw_ref[...]
    tmod = t_ref[...] % 16
    lane16 = jax.lax.broadcasted_iota(jnp.int32, (_B, 16), 1)
    tv = jnp.sum(jnp.where(lane16 == tmod, tw, 0.0), axis=1, keepdims=True)

    m = m_ref[...]
    s = s_ref[...]
    out_t = _S * tv - _S * _M
    s_c = s - jnp.exp(_S * tv - _S * m) + jnp.exp(out_t - _S * m)
    loss = _S * m + jnp.log(s_c) - out_t
    out_ref[...] = (jnp.sum(loss) / _B).reshape(1, 1)


def kernel(cos_theta, cos_theta_aux, target):
    B, C = cos_theta.shape
    t32 = target.astype(jnp.int32)
    m_tc, s_tc = _tc_main(cos_theta)
    tw = _make_sc()(cos_theta, t32)

    out = pl.pallas_call(
        _combine_kernel,
        grid=(1,),
        in_specs=[
            pl.BlockSpec((B, 1), lambda i: (0, 0)),
            pl.BlockSpec((B, 1), lambda i: (0, 0)),
            pl.BlockSpec((B, 16), lambda i: (0, 0)),
            pl.BlockSpec((B, 1), lambda i: (0, 0)),
        ],
        out_specs=pl.BlockSpec((1, 1), lambda i: (0, 0)),
        out_shape=jax.ShapeDtypeStruct((1, 1), jnp.float32),
    )(m_tc, s_tc, tw, t32.reshape(B, 1))
    return out[0, 0]


# speedup vs baseline: 1.6409x; 1.0026x over previous
import functools

import jax
import jax.numpy as jnp
from jax import lax
from jax.experimental import pallas as pl
from jax.experimental.pallas import tpu as pltpu
from jax.experimental.pallas import tpu_sc as plsc

_S = 3.0
_M = 0.2

_B, _C = 1024, 100000


def _tc_kernel(x_ref, m_ref, s_ref, *, nblk, blk):
    k = pl.program_id(0)

    @pl.when(k == 0)
    def _init():
        m_ref[...] = jnp.full((_B, 1), -jnp.inf, jnp.float32)
        s_ref[...] = jnp.zeros((_B, 1), jnp.float32)

    def update(x):
        bm = jnp.max(x, axis=1, keepdims=True)
        bs = jnp.sum(jnp.exp(_S * x - _S * bm), axis=1, keepdims=True)
        m_old = m_ref[...]
        m_new = jnp.maximum(m_old, bm)
        s_ref[...] = s_ref[...] * jnp.exp(_S * (m_old - m_new)) + bs * jnp.exp(
            _S * (bm - m_new)
        )
        m_ref[...] = m_new

    @pl.when(k < nblk - 1)
    def _full_block():
        update(x_ref[...])

    @pl.when(k == nblk - 1)
    def _tail_block():
        cols = jax.lax.broadcasted_iota(jnp.int32, (_B, blk), 1) + k * blk
        update(jnp.where(cols < _C, x_ref[...], -jnp.inf))


def _tc_main(cos_theta):
    blk = 4096
    nblk = pl.cdiv(_C, blk)
    return pl.pallas_call(
        functools.partial(_tc_kernel, nblk=nblk, blk=blk),
        grid=(nblk,),
        in_specs=[pl.BlockSpec((_B, blk), lambda k: (0, k))],
        out_specs=[
            pl.BlockSpec((_B, 1), lambda k: (0, 0)),
            pl.BlockSpec((_B, 1), lambda k: (0, 0)),
        ],
        out_shape=[
            jax.ShapeDtypeStruct((_B, 1), jnp.float32),
            jax.ShapeDtypeStruct((_B, 1), jnp.float32),
        ],
    )(cos_theta)


def _make_sc():
    info = plsc.get_sparse_core_info()
    nc, ns = info.num_cores, info.num_subcores
    nw = nc * ns
    g_per_w = _B // nw
    mesh = plsc.VectorSubcoreMesh(core_axis_name="c", subcore_axis_name="s")

    @functools.partial(
        pl.kernel,
        mesh=mesh,
        out_type=jax.ShapeDtypeStruct((_B, 16), jnp.float32),
        scratch_types=[
            pltpu.VMEM((g_per_w,), jnp.int32),
            pltpu.VMEM((128,), jnp.float32),
            pltpu.VMEM((g_per_w, 16), jnp.float32),
        ],
    )
    def sc_kernel(x_hbm, t_hbm, tv_hbm, tbuf, gbuf, wout):
        wid = lax.axis_index("s") * nc + lax.axis_index("c")
        gbase = pl.multiple_of(wid * g_per_w, 8)
        pltpu.sync_copy(t_hbm.at[pl.ds(gbase, g_per_w)], tbuf)

        for r in range(g_per_w):
            t = tbuf[pl.ds((r // 16) * 16, 16)][r % 16]
            c0 = pl.multiple_of((t // 128) * 128, 128)
            pltpu.sync_copy(x_hbm.at[gbase + r, pl.ds(c0, 128)], gbuf)
            j16 = (t - c0) // 16
            w = jnp.zeros((16,), jnp.float32)
            for j in range(8):
                w = jnp.where(j16 == j, gbuf[pl.ds(j * 16, 16)], w)
            wout[r, :] = w
        pltpu.sync_copy(wout, tv_hbm.at[pl.ds(gbase, g_per_w)])

    return sc_kernel


def _combine_kernel(m_ref, s_ref, tw_ref, t_ref, out_ref):
    tw = tw_ref[...]
    tmod = t_ref[...] % 16
    lane16 = jax.lax.broadcasted_iota(jnp.int32, (_B, 16), 1)
    tv = jnp.sum(jnp.where(lane16 == tmod, tw, 0.0), axis=1, keepdims=True)

    m = m_ref[...]
    s = s_ref[...]
    out_t = _S * tv - _S * _M
    s_c = s - jnp.exp(_S * tv - _S * m) + jnp.exp(out_t - _S * m)
    loss = _S * m + jnp.log(s_c) - out_t
    out_ref[...] = (jnp.sum(loss) / _B).reshape(1, 1)


def kernel(cos_theta, cos_theta_aux, target):
    B, C = cos_theta.shape
    t32 = target.astype(jnp.int32)
    m_tc, s_tc = _tc_main(cos_theta)
    tw = _make_sc()(cos_theta, t32)

    out = pl.pallas_call(
        _combine_kernel,
        grid=(1,),
        in_specs=[
            pl.BlockSpec((B, 1), lambda i: (0, 0)),
            pl.BlockSpec((B, 1), lambda i: (0, 0)),
            pl.BlockSpec((B, 16), lambda i: (0, 0)),
            pl.BlockSpec((B, 1), lambda i: (0, 0)),
        ],
        out_specs=pl.BlockSpec((1, 1), lambda i: (0, 0)),
        out_shape=jax.ShapeDtypeStruct((1, 1), jnp.float32),
    )(m_tc, s_tc, tw, t32.reshape(B, 1))
    return out[0, 0]
